# baseline (speedup 1.0000x reference)
import functools

import jax
import jax.numpy as jnp
from jax import lax
from jax.experimental import pallas as pl
from jax.experimental.pallas import tpu as pltpu
from jax.experimental.pallas import tpu_sc as plsc

_N = 2048
_H = 16
_LANES = 16
_PITCH = 4160


def _sc_body(bt_hbm, out_hbm, btv, dsh, sem):
    nc = 2
    wid = lax.axis_index("s") * nc + lax.axis_index("c")
    h = wid // 2
    halfbit = wid % 2
    row0 = halfbit * (_N // 2)

    pltpu.sync_copy(bt_hbm.at[h], btv)

    jlo = (1 - halfbit) * 63
    jhi = 192 + (1 - halfbit) * 64
    lanes = lax.iota(jnp.int32, _LANES)
    for s in range(8):
        @plsc.parallel_loop(jlo, jhi, unroll=8)
        def _chunk(j, s=s):
            base = j * _LANES
            idx = jnp.clip(base + lanes + (s - (_N - 1)), -1000, 999) + 1000
            dsh[pl.ds(s * _PITCH + base, _LANES)] = plsc.load_gather(btv, [idx])

    def blk_body(blk, carry):
        i_base = row0 + 8 * blk
        for tt in range(8):
            s = 7 - tt
            i = i_base + tt
            v = (_N - 1) - i
            off = pl.multiple_of(v + (_PITCH - 1) * s, 8)
            pltpu.async_copy(
                dsh.at[pl.ds(off, _N)], out_hbm.at[0, h, i, :], sem
            )

        @pl.when(blk > 0)
        def _drain_prev():
            for tt in range(8):
                pltpu.make_async_copy(
                    dsh.at[pl.ds(0, _N)], out_hbm.at[0, h, i_base + tt, :], sem
                ).wait()

        return carry

    lax.fori_loop(0, (_N // 2) // 8, blk_body, 0)
    for tt in range(8):
        pltpu.make_async_copy(
            dsh.at[pl.ds(0, _N)], out_hbm.at[0, h, row0 + tt, :], sem
        ).wait()


def kernel(x, bias_values):
    del x
    bt = jnp.transpose(bias_values)
    bt = jnp.pad(bt, ((0, 0), (0, 48)))

    mesh = plsc.VectorSubcoreMesh(core_axis_name="c", subcore_axis_name="s")
    run = functools.partial(
        pl.kernel,
        out_type=jax.ShapeDtypeStruct((1, _H, _N, _N), jnp.float32),
        mesh=mesh,
        scratch_types=[
            pltpu.VMEM((_N,), jnp.float32),
            pltpu.VMEM((8 * _PITCH,), jnp.float32),
            pltpu.SemaphoreType.DMA,
        ],
        compiler_params=pltpu.CompilerParams(
            needs_layout_passes=False,
            use_tc_tiling_on_sc=False,
        ),
    )
    return run(_sc_body)(bt)
